# baseline (speedup 1.0000x reference)
import functools

import numpy as np
import jax
import jax.numpy as jnp
from jax import lax
from jax.experimental import pallas as pl
from jax.experimental.pallas import tpu as pltpu
from jax.experimental.pallas import tpu_sc as plsc

N = 10000
E = 160000
F = 128
DE = 4
DSC = 8
H1 = 64
H2 = 64
FOUT = 128
NUM_NEIGHBORS = 16.0
MIXING_ANGLE = np.pi / 8.0

NC = 2
NS = 16
NW = NC * NS
CHUNK = 128
E_PAD = 163840
NPH = 4
EPH = E_PAD // NPH
CPW = 10
CPW_PAD = 16
EPW = CPW * CHUNK
ROWS_PER_SUB = 624
ROWS_TAIL = N - NS * ROWS_PER_SUB

_COS = float(np.cos(MIXING_ANGLE))
_SIN = float(np.sin(MIXING_ANGLE))
_EDGE_SCALE = _SIN / (np.sqrt(H2) * np.sqrt(NUM_NEIGHBORS))

_SC_MESH = plsc.VectorSubcoreMesh(
    core_axis_name="c", subcore_axis_name="s", num_cores=NC, num_subcores=NS
)


def _lin_body(x_ref, w_ref, feat_ref, self_ref):
    t = jnp.dot(x_ref[...], w_ref[...], preferred_element_type=jnp.float32)
    feat_ref[...] = t[:, :F]
    self_ref[...] = t[:, F:] * (0.5 * _COS)


_LIN_ROWS = 2000


def _linear(node_input, W_lin):
    return pl.pallas_call(
        _lin_body,
        grid=(N // _LIN_ROWS,),
        in_specs=[
            pl.BlockSpec((_LIN_ROWS, F), lambda i: (i, 0)),
            pl.BlockSpec((F, F + FOUT), lambda i: (0, 0)),
        ],
        out_specs=[
            pl.BlockSpec((_LIN_ROWS, F), lambda i: (i, 0)),
            pl.BlockSpec((_LIN_ROWS, FOUT), lambda i: (i, 0)),
        ],
        out_shape=[
            jax.ShapeDtypeStruct((N, F), jnp.float32),
            jax.ShapeDtypeStruct((N, FOUT), jnp.float32),
        ],
    )(node_input, W_lin)


@functools.partial(
    pl.kernel,
    out_type=jax.ShapeDtypeStruct((EPH, F), jnp.float32),
    mesh=_SC_MESH,
    scratch_types=[
        pltpu.VMEM((CPW_PAD, CHUNK), jnp.int32),
        pltpu.VMEM((CHUNK, F), jnp.float32),
        pltpu.VMEM((CHUNK, F), jnp.float32),
        pltpu.SemaphoreType.DMA,
        pltpu.SemaphoreType.DMA,
        pltpu.SemaphoreType.DMA,
        pltpu.SemaphoreType.DMA,
    ],
)
def _gather(feat_hbm, src3d_hbm, out_hbm, idx_all, bufa, bufb,
            sem_ga, sem_gb, sem_oa, sem_ob):
    c = lax.axis_index("c")
    s = lax.axis_index("s")
    w = c * NS + s
    base = w * EPW

    def fire_gather(buf, sem, i):
        pltpu.async_copy(feat_hbm.at[idx_all.at[i]], buf, sem)

    def drain_gather(buf, sem):
        pltpu.make_async_copy(feat_hbm.at[idx_all.at[0]], buf, sem).wait()

    def fire_out(buf, sem, i):
        pltpu.async_copy(buf, out_hbm.at[pl.ds(base + i * CHUNK, CHUNK)], sem)

    def drain_out(buf, sem):
        pltpu.make_async_copy(buf, out_hbm.at[pl.ds(0, CHUNK)], sem).wait()

    pltpu.sync_copy(src3d_hbm.at[w], idx_all)
    fire_gather(bufa, sem_ga, 0)

    def body(g, carry):
        @pl.when(g > 0)
        def _():
            drain_out(bufb, sem_ob)
        fire_gather(bufb, sem_gb, 2 * g + 1)
        drain_gather(bufa, sem_ga)
        fire_out(bufa, sem_oa, 2 * g)
        drain_gather(bufb, sem_gb)
        fire_out(bufb, sem_ob, 2 * g + 1)
        drain_out(bufa, sem_oa)
        fire_gather(bufa, sem_ga, 2 * g + 2)
        return carry

    lax.fori_loop(0, CPW // 2 - 1, body, 0)
    gl = CPW // 2 - 1
    drain_out(bufb, sem_ob)
    fire_gather(bufb, sem_gb, 2 * gl + 1)
    drain_gather(bufa, sem_ga)
    fire_out(bufa, sem_oa, 2 * gl)
    drain_gather(bufb, sem_gb)
    fire_out(bufb, sem_ob, 2 * gl + 1)
    drain_out(bufa, sem_oa)
    drain_out(bufb, sem_ob)


_EB = 8192

_DN_T = (((0,), (0,)), ((), ()))


def _edge_body(attr_ref, ef_ref, w1_ref, w2_ref, wtp_ref, wout_ref, out_ref):
    attr = attr_ref[...]
    h1t = jax.nn.gelu(lax.dot_general(w1_ref[...], attr, _DN_T,
                                      preferred_element_type=jnp.float32))
    h2t = jax.nn.gelu(lax.dot_general(w2_ref[...], h1t, _DN_T,
                                      preferred_element_type=jnp.float32))
    ef = ef_ref[...]
    mid = jnp.concatenate(
        [lax.dot_general(h2t * attr[DSC + j:DSC + j + 1, :],
                         wtp_ref[:, j * F:(j + 1) * F], _DN_T,
                         preferred_element_type=jnp.float32) * ef
         for j in range(DE)],
        axis=1)
    out_ref[...] = jnp.dot(mid, wout_ref[...],
                           preferred_element_type=jnp.float32) * _EDGE_SCALE


def _edge_compute(phase, attr_t, edge_features, mlp_w1, mlp_w2, wtp2d, wout_perm):
    nb = EPH // _EB
    off = phase * nb
    return pl.pallas_call(
        _edge_body,
        grid=(nb,),
        in_specs=[
            pl.BlockSpec((DSC + DE, _EB), lambda i: (0, i + off)),
            pl.BlockSpec((_EB, F), lambda i: (i, 0)),
            pl.BlockSpec((DSC + DE, H1), lambda i: (0, 0)),
            pl.BlockSpec((H1, H2), lambda i: (0, 0)),
            pl.BlockSpec((H2, DE * F), lambda i: (0, 0)),
            pl.BlockSpec((DE * F, FOUT), lambda i: (0, 0)),
        ],
        out_specs=pl.BlockSpec((_EB, FOUT), lambda i: (i, 0)),
        out_shape=jax.ShapeDtypeStruct((EPH, FOUT), jnp.float32),
    )(attr_t, edge_features, mlp_w1, mlp_w2, wtp2d, wout_perm)


def _make_scatter(seed3d):
    @functools.partial(
        pl.kernel,
        out_type=jax.ShapeDtypeStruct((NC, N, FOUT), jnp.float32),
        mesh=_SC_MESH,
        scratch_types=[
            pltpu.VMEM((CPW_PAD, CHUNK), jnp.int32),
            pltpu.VMEM((CHUNK, FOUT), jnp.float32),
            pltpu.VMEM((CHUNK, FOUT), jnp.float32),
            pltpu.VMEM_SHARED((N, FOUT), jnp.float32),
            pltpu.SemaphoreType.DMA,
            pltpu.SemaphoreType.DMA,
            pltpu.SemaphoreType.DMA,
            pltpu.SemaphoreType.DMA,
        ],
    )
    def _scatter(edge_out_hbm, dst3d_hbm, seed_hbm, part_hbm,
                 idx_all, bufa, bufb, acc_sh, sem_ia, sem_ib, sem_sa, sem_sb):
        c = lax.axis_index("c")
        s = lax.axis_index("s")
        w = c * NS + s
        base = w * EPW

        def fire_in(buf, sem, i):
            pltpu.async_copy(edge_out_hbm.at[pl.ds(base + i * CHUNK, CHUNK)], buf, sem)

        def drain_in(buf, sem):
            pltpu.make_async_copy(edge_out_hbm.at[pl.ds(0, CHUNK)], buf, sem).wait()

        def fire_scatter(buf, sem, i):
            pltpu.async_copy(buf, acc_sh.at[idx_all.at[i]], sem, add=True)

        def drain_scatter(buf, sem):
            pltpu.make_async_copy(buf, acc_sh.at[idx_all.at[0]], sem).wait()

        seed = seed_hbm.at[c] if seed3d else seed_hbm
        r0 = s * ROWS_PER_SUB
        pltpu.sync_copy(seed.at[pl.ds(r0, ROWS_PER_SUB)], acc_sh.at[pl.ds(r0, ROWS_PER_SUB)])
        @pl.when(s == 0)
        def _():
            pltpu.sync_copy(seed.at[pl.ds(NS * ROWS_PER_SUB, ROWS_TAIL)],
                            acc_sh.at[pl.ds(NS * ROWS_PER_SUB, ROWS_TAIL)])
        pltpu.sync_copy(dst3d_hbm.at[w], idx_all)
        plsc.subcore_barrier()
        fire_in(bufa, sem_ia, 0)

        def body(g, carry):
            @pl.when(g > 0)
            def _():
                drain_scatter(bufb, sem_sb)
            fire_in(bufb, sem_ib, 2 * g + 1)
            drain_in(bufa, sem_ia)
            fire_scatter(bufa, sem_sa, 2 * g)
            drain_in(bufb, sem_ib)
            fire_scatter(bufb, sem_sb, 2 * g + 1)
            drain_scatter(bufa, sem_sa)
            fire_in(bufa, sem_ia, 2 * g + 2)
            return carry

        lax.fori_loop(0, CPW // 2 - 1, body, 0)
        gl = CPW // 2 - 1
        drain_scatter(bufb, sem_sb)
        fire_in(bufb, sem_ib, 2 * gl + 1)
        drain_in(bufa, sem_ia)
        fire_scatter(bufa, sem_sa, 2 * gl)
        drain_in(bufb, sem_ib)
        fire_scatter(bufb, sem_sb, 2 * gl + 1)
        drain_scatter(bufa, sem_sa)
        drain_scatter(bufb, sem_sb)

        plsc.subcore_barrier()
        pltpu.sync_copy(acc_sh.at[pl.ds(r0, ROWS_PER_SUB)], part_hbm.at[c, pl.ds(r0, ROWS_PER_SUB)])
        @pl.when(s == 0)
        def _():
            pltpu.sync_copy(acc_sh.at[pl.ds(NS * ROWS_PER_SUB, ROWS_TAIL)],
                            part_hbm.at[c, pl.ds(NS * ROWS_PER_SUB, ROWS_TAIL)])

    return _scatter


_scatter_first = _make_scatter(False)
_scatter_chain = _make_scatter(True)


def _combine_body(p_ref, out_ref):
    out_ref[...] = p_ref[0] + p_ref[1]


def _combine(p):
    return pl.pallas_call(
        _combine_body,
        grid=(N // _LIN_ROWS,),
        in_specs=[
            pl.BlockSpec((NC, _LIN_ROWS, FOUT), lambda i: (0, i, 0)),
        ],
        out_specs=pl.BlockSpec((_LIN_ROWS, FOUT), lambda i: (i, 0)),
        out_shape=jax.ShapeDtypeStruct((N, FOUT), jnp.float32),
    )(p)


def kernel(node_input, edge_attr, edge_scalar_attr, W_lin, mlp_w1, mlp_w2, w_tp, W_out, edge_src, edge_dst):
    wtp2d = w_tp.transpose(0, 2, 1).reshape(H2, DE * F)
    wout_perm = W_out.reshape(F, DE, FOUT).transpose(1, 0, 2).reshape(DE * F, FOUT)
    mlp_w1 = jnp.concatenate([mlp_w1, jnp.zeros((DE, H1), jnp.float32)])
    npad = E_PAD - E
    pad_idx = (jnp.arange(npad, dtype=jnp.int32) * 37) % N
    edge_src = jnp.concatenate([edge_src.astype(jnp.int32), pad_idx])
    edge_dst = jnp.concatenate([edge_dst.astype(jnp.int32), pad_idx])
    attr = jnp.concatenate(
        [jnp.concatenate([edge_scalar_attr, edge_attr], axis=1),
         jnp.zeros((npad, DSC + DE), jnp.float32)])
    attr_t = attr.T

    def idx3d(idx):
        main = idx.reshape(NPH, NW, CPW, CHUNK)
        pad = jnp.zeros((NPH, NW, CPW_PAD - CPW, CHUNK), dtype=idx.dtype)
        return jnp.concatenate([main, pad], axis=2)

    src3d = idx3d(edge_src)
    dst3d = idx3d(edge_dst)

    node_features, self_half = _linear(node_input, W_lin)
    seed = self_half
    for p in range(NPH):
        ef_p = _gather(node_features, src3d[p])
        eo_p = _edge_compute(p, attr_t, ef_p, mlp_w1, mlp_w2, wtp2d, wout_perm)
        scatter = _scatter_first if p == 0 else _scatter_chain
        seed = scatter(eo_p, dst3d[p], seed)
    return _combine(seed)


# speedup vs baseline: 1.0191x; 1.0191x over previous
import functools

import numpy as np
import jax
import jax.numpy as jnp
from jax import lax
from jax.experimental import pallas as pl
from jax.experimental.pallas import tpu as pltpu
from jax.experimental.pallas import tpu_sc as plsc

N = 10000
E = 160000
F = 128
DE = 4
DSC = 8
H1 = 64
H2 = 64
FOUT = 128
NUM_NEIGHBORS = 16.0
MIXING_ANGLE = np.pi / 8.0

NC = 2
NS = 16
NW = NC * NS
CHUNK = 128
E_PAD = 163840
NPH = 2
EPH = E_PAD // NPH
CPW = 20
CPW_PAD = 24
EPW = CPW * CHUNK
PAIR = 2 * CHUNK
NPAIR = CPW // 2
ROWS_PER_SUB = 624
ROWS_TAIL = N - NS * ROWS_PER_SUB

_COS = float(np.cos(MIXING_ANGLE))
_SIN = float(np.sin(MIXING_ANGLE))
_EDGE_SCALE = _SIN / (np.sqrt(H2) * np.sqrt(NUM_NEIGHBORS))

_SC_MESH = plsc.VectorSubcoreMesh(
    core_axis_name="c", subcore_axis_name="s", num_cores=NC, num_subcores=NS
)


def _lin_body(x_ref, w_ref, feat_ref, self_ref):
    t = jnp.dot(x_ref[...], w_ref[...], preferred_element_type=jnp.float32)
    feat_ref[...] = t[:, :F]
    self_ref[...] = t[:, F:] * (0.25 * _COS)


_LIN_ROWS = 2000


def _linear(node_input, W_lin):
    return pl.pallas_call(
        _lin_body,
        grid=(N // _LIN_ROWS,),
        in_specs=[
            pl.BlockSpec((_LIN_ROWS, F), lambda i: (i, 0)),
            pl.BlockSpec((F, F + FOUT), lambda i: (0, 0)),
        ],
        out_specs=[
            pl.BlockSpec((_LIN_ROWS, F), lambda i: (i, 0)),
            pl.BlockSpec((_LIN_ROWS, FOUT), lambda i: (i, 0)),
        ],
        out_shape=[
            jax.ShapeDtypeStruct((N, F), jnp.float32),
            jax.ShapeDtypeStruct((N, FOUT), jnp.float32),
        ],
    )(node_input, W_lin)


@functools.partial(
    pl.kernel,
    out_type=jax.ShapeDtypeStruct((EPH, F), jnp.float32),
    mesh=_SC_MESH,
    scratch_types=[
        pltpu.VMEM((CPW_PAD, CHUNK), jnp.int32),
        pltpu.VMEM((PAIR, F), jnp.float32),
        pltpu.VMEM((PAIR, F), jnp.float32),
        pltpu.SemaphoreType.DMA,
        pltpu.SemaphoreType.DMA,
        pltpu.SemaphoreType.DMA,
        pltpu.SemaphoreType.DMA,
    ],
)
def _gather(feat_hbm, src3d_hbm, out_hbm, idx_all, bufa, bufb,
            sem_ga, sem_gb, sem_oa, sem_ob):
    c = lax.axis_index("c")
    s = lax.axis_index("s")
    w = c * NS + s
    base = w * EPW

    def fire_gathers(buf, sem, u):
        for b in range(2):
            pltpu.async_copy(feat_hbm.at[idx_all.at[2 * u + b]],
                             buf.at[pl.ds(b * CHUNK, CHUNK)], sem)

    def drain_gathers(buf, sem):
        for b in range(2):
            pltpu.make_async_copy(feat_hbm.at[idx_all.at[0]],
                                  buf.at[pl.ds(b * CHUNK, CHUNK)], sem).wait()

    def fire_out(buf, sem, u):
        pltpu.async_copy(buf, out_hbm.at[pl.ds(base + u * PAIR, PAIR)], sem)

    def drain_out(buf, sem):
        pltpu.make_async_copy(buf, out_hbm.at[pl.ds(0, PAIR)], sem).wait()

    pltpu.sync_copy(src3d_hbm.at[w], idx_all)
    fire_gathers(bufa, sem_ga, 0)

    def body(g, carry):
        @pl.when(g > 0)
        def _():
            drain_out(bufb, sem_ob)
        fire_gathers(bufb, sem_gb, 2 * g + 1)
        drain_gathers(bufa, sem_ga)
        fire_out(bufa, sem_oa, 2 * g)
        drain_gathers(bufb, sem_gb)
        fire_out(bufb, sem_ob, 2 * g + 1)
        drain_out(bufa, sem_oa)
        fire_gathers(bufa, sem_ga, 2 * g + 2)
        return carry

    lax.fori_loop(0, NPAIR // 2 - 1, body, 0)
    gl = NPAIR // 2 - 1
    drain_out(bufb, sem_ob)
    fire_gathers(bufb, sem_gb, 2 * gl + 1)
    drain_gathers(bufa, sem_ga)
    fire_out(bufa, sem_oa, 2 * gl)
    drain_gathers(bufb, sem_gb)
    fire_out(bufb, sem_ob, 2 * gl + 1)
    drain_out(bufa, sem_oa)
    drain_out(bufb, sem_ob)


_EB = 8192

_DN_T = (((0,), (0,)), ((), ()))


def _edge_body(attr_ref, ef_ref, w1_ref, w2_ref, wtp_ref, wout_ref, out_ref):
    attr = attr_ref[...]
    h1t = jax.nn.gelu(lax.dot_general(w1_ref[...], attr, _DN_T,
                                      preferred_element_type=jnp.float32))
    h2t = jax.nn.gelu(lax.dot_general(w2_ref[...], h1t, _DN_T,
                                      preferred_element_type=jnp.float32))
    ef = ef_ref[...]
    mid = jnp.concatenate(
        [lax.dot_general(h2t * attr[DSC + j:DSC + j + 1, :],
                         wtp_ref[:, j * F:(j + 1) * F], _DN_T,
                         preferred_element_type=jnp.float32) * ef
         for j in range(DE)],
        axis=1)
    out_ref[...] = jnp.dot(mid, wout_ref[...],
                           preferred_element_type=jnp.float32)


def _edge_compute(phase, attr_t, edge_features, mlp_w1, mlp_w2, wtp2d, wout_perm):
    nb = EPH // _EB
    off = phase * nb
    return pl.pallas_call(
        _edge_body,
        grid=(nb,),
        in_specs=[
            pl.BlockSpec((DSC + DE, _EB), lambda i: (0, i + off)),
            pl.BlockSpec((_EB, F), lambda i: (i, 0)),
            pl.BlockSpec((DSC + DE, H1), lambda i: (0, 0)),
            pl.BlockSpec((H1, H2), lambda i: (0, 0)),
            pl.BlockSpec((H2, DE * F), lambda i: (0, 0)),
            pl.BlockSpec((DE * F, FOUT), lambda i: (0, 0)),
        ],
        out_specs=pl.BlockSpec((_EB, FOUT), lambda i: (i, 0)),
        out_shape=jax.ShapeDtypeStruct((EPH, FOUT), jnp.float32),
    )(attr_t, edge_features, mlp_w1, mlp_w2, wtp2d, wout_perm)


@functools.partial(
    pl.kernel,
    out_type=jax.ShapeDtypeStruct((NC, N, FOUT), jnp.float32),
    mesh=_SC_MESH,
    scratch_types=[
        pltpu.VMEM((CPW_PAD, CHUNK), jnp.int32),
        pltpu.VMEM((CHUNK, FOUT), jnp.float32),
        pltpu.VMEM((CHUNK, FOUT), jnp.float32),
        pltpu.VMEM_SHARED((N, FOUT), jnp.float32),
        pltpu.SemaphoreType.DMA,
        pltpu.SemaphoreType.DMA,
        pltpu.SemaphoreType.DMA,
        pltpu.SemaphoreType.DMA,
    ],
)
def _scatter(edge_out_hbm, dst3d_hbm, self_hbm, part_hbm,
             idx_all, bufa, bufb, acc_sh, sem_ia, sem_ib, sem_sa, sem_sb):
    c = lax.axis_index("c")
    s = lax.axis_index("s")
    w = c * NS + s
    base = w * EPW

    def fire_in(buf, sem, i):
        pltpu.async_copy(edge_out_hbm.at[pl.ds(base + i * CHUNK, CHUNK)], buf, sem)

    def drain_in(buf, sem):
        pltpu.make_async_copy(edge_out_hbm.at[pl.ds(0, CHUNK)], buf, sem).wait()

    def fire_scatter(buf, sem, i):
        pltpu.async_copy(buf, acc_sh.at[idx_all.at[i]], sem, add=True)

    def drain_scatter(buf, sem):
        pltpu.make_async_copy(buf, acc_sh.at[idx_all.at[0]], sem).wait()

    r0 = s * ROWS_PER_SUB
    pltpu.sync_copy(self_hbm.at[pl.ds(r0, ROWS_PER_SUB)], acc_sh.at[pl.ds(r0, ROWS_PER_SUB)])
    @pl.when(s == 0)
    def _():
        pltpu.sync_copy(self_hbm.at[pl.ds(NS * ROWS_PER_SUB, ROWS_TAIL)],
                        acc_sh.at[pl.ds(NS * ROWS_PER_SUB, ROWS_TAIL)])
    pltpu.sync_copy(dst3d_hbm.at[w], idx_all)
    plsc.subcore_barrier()
    fire_in(bufa, sem_ia, 0)

    def body(g, carry):
        @pl.when(g > 0)
        def _():
            drain_scatter(bufb, sem_sb)
        fire_in(bufb, sem_ib, 2 * g + 1)
        drain_in(bufa, sem_ia)
        fire_scatter(bufa, sem_sa, 2 * g)
        drain_in(bufb, sem_ib)
        fire_scatter(bufb, sem_sb, 2 * g + 1)
        drain_scatter(bufa, sem_sa)
        fire_in(bufa, sem_ia, 2 * g + 2)
        return carry

    lax.fori_loop(0, CPW // 2 - 1, body, 0)
    gl = CPW // 2 - 1
    drain_scatter(bufb, sem_sb)
    fire_in(bufb, sem_ib, 2 * gl + 1)
    drain_in(bufa, sem_ia)
    fire_scatter(bufa, sem_sa, 2 * gl)
    drain_in(bufb, sem_ib)
    fire_scatter(bufb, sem_sb, 2 * gl + 1)
    drain_scatter(bufa, sem_sa)
    drain_scatter(bufb, sem_sb)

    plsc.subcore_barrier()
    pltpu.sync_copy(acc_sh.at[pl.ds(r0, ROWS_PER_SUB)], part_hbm.at[c, pl.ds(r0, ROWS_PER_SUB)])
    @pl.when(s == 0)
    def _():
        pltpu.sync_copy(acc_sh.at[pl.ds(NS * ROWS_PER_SUB, ROWS_TAIL)],
                        part_hbm.at[c, pl.ds(NS * ROWS_PER_SUB, ROWS_TAIL)])


def _combine_body(pa_ref, pb_ref, out_ref):
    out_ref[...] = (pa_ref[0] + pa_ref[1]) + (pb_ref[0] + pb_ref[1])


def _combine(pa, pb):
    return pl.pallas_call(
        _combine_body,
        grid=(N // _LIN_ROWS,),
        in_specs=[
            pl.BlockSpec((NC, _LIN_ROWS, FOUT), lambda i: (0, i, 0)),
            pl.BlockSpec((NC, _LIN_ROWS, FOUT), lambda i: (0, i, 0)),
        ],
        out_specs=pl.BlockSpec((_LIN_ROWS, FOUT), lambda i: (i, 0)),
        out_shape=jax.ShapeDtypeStruct((N, FOUT), jnp.float32),
    )(pa, pb)


def kernel(node_input, edge_attr, edge_scalar_attr, W_lin, mlp_w1, mlp_w2, w_tp, W_out, edge_src, edge_dst):
    wtp2d = w_tp.transpose(0, 2, 1).reshape(H2, DE * F)
    wout_perm = (W_out.reshape(F, DE, FOUT).transpose(1, 0, 2).reshape(DE * F, FOUT)
                 * _EDGE_SCALE)
    mlp_w1 = jnp.concatenate([mlp_w1, jnp.zeros((DE, H1), jnp.float32)])
    npad = E_PAD - E
    pad_idx = (jnp.arange(npad, dtype=jnp.int32) * 37) % N
    edge_src = jnp.concatenate([edge_src.astype(jnp.int32), pad_idx])
    edge_dst = jnp.concatenate([edge_dst.astype(jnp.int32), pad_idx])
    attr = jnp.concatenate(
        [jnp.concatenate([edge_scalar_attr, edge_attr], axis=1),
         jnp.zeros((npad, DSC + DE), jnp.float32)])
    attr_t = attr.T

    def idx3d(idx):
        main = idx.reshape(NPH, NW, CPW, CHUNK)
        pad = jnp.zeros((NPH, NW, CPW_PAD - CPW, CHUNK), dtype=idx.dtype)
        return jnp.concatenate([main, pad], axis=2)

    src3d = idx3d(edge_src)
    dst3d = idx3d(edge_dst)

    node_features, self_q = _linear(node_input, W_lin)
    parts = []
    for p in range(NPH):
        ef_p = _gather(node_features, src3d[p])
        eo_p = _edge_compute(p, attr_t, ef_p, mlp_w1, mlp_w2, wtp2d, wout_perm)
        parts.append(_scatter(eo_p, dst3d[p], self_q))
    return _combine(parts[0], parts[1])
